# baseline (speedup 1.0000x reference)
import functools

import jax
import jax.numpy as jnp
from jax import lax
from jax.experimental import pallas as pl
from jax.experimental.pallas import tpu as pltpu
from jax.experimental.pallas import tpu_sc as plsc

_G = 0.1
_CHUNK = 128
_NC = 2
_NS = 16
_NW = _NC * _NS


_GRP = 4
_GE = _GRP * _CHUNK
_XTRA = 2 * _GRP


def _pad_edges(src, dst, w, mult):
    e = src.shape[0]
    ep = -(-e // mult) * mult
    src = jnp.concatenate([src, jnp.zeros((ep - e,), jnp.int32)])
    dst = jnp.concatenate([dst, jnp.zeros((ep - e,), jnp.int32)])
    w = jnp.concatenate([w, jnp.zeros((ep - e + _XTRA * _CHUNK,),
                                      jnp.float32)])
    idx = jnp.stack([src.reshape(-1, _CHUNK), dst.reshape(-1, _CHUNK)],
                    axis=1).reshape(-1, _CHUNK)
    idx = jnp.concatenate(
        [idx, jnp.zeros((2 * _XTRA, _CHUNK), jnp.int32)])
    return idx, w, ep


@functools.lru_cache(maxsize=None)
def _make_spmm(n, d, epa, epk, nc=_NC):
    nw = nc * _NS
    ga = epa // (nw * _GE)
    gk = epk // (nw * _GE)
    rps = (n // _NS) // 8 * 8
    tail = n - _NS * rps
    mesh = plsc.VectorSubcoreMesh(core_axis_name="c", subcore_axis_name="s",
                                  num_cores=nc)

    @functools.partial(
        pl.kernel,
        mesh=mesh,
        compiler_params=pltpu.CompilerParams(use_tc_tiling_on_sc=False),
        out_type=(jax.ShapeDtypeStruct((nc * n, d), jnp.float32),
                  jax.ShapeDtypeStruct((nc * n, d), jnp.float32)),
        scratch_types=[
            pltpu.VMEM((2 * _GRP * 2, _CHUNK), jnp.int32),
            pltpu.VMEM((2, _GE), jnp.float32),
            pltpu.VMEM((2, _GE, d), jnp.float32),
            pltpu.VMEM((208, d), jnp.float32),
            pltpu.VMEM_SHARED((n, d), jnp.float32),
            pltpu.SemaphoreType.DMA,
            pltpu.SemaphoreType.DMA,
            pltpu.SemaphoreType.DMA,
        ],
    )
    def spmm(sup, idxA, wA, idxK, wK,
             outA, outK, idxb, wbuf, rows, sbuf, acc, semI, semG, semS):
        c = lax.axis_index("c")
        s = lax.axis_index("s")
        wid = c * _NS + s

        r0 = pl.multiple_of(s * rps, 8)
        _SB = 208
        zv = jnp.zeros((16,), jnp.float32)

        def zero_sbuf():
            def zb(i, c2):
                for gcol in range(d // 16):
                    sbuf[i, pl.ds(gcol * 16, 16)] = zv
                return c2

            lax.fori_loop(0, _SB, zb, 0)

        zero_sbuf()

        def scale(p):
            def sbody(e16, c2):
                e0 = pl.multiple_of(e16 * 16, 16)
                wvec = wbuf[p, pl.ds(e0, 16)]
                for lane in range(16):
                    e = e0 + lane
                    w = wvec[lane]
                    for g in range(d // 16):
                        sl = pl.ds(g * 16, 16)
                        rows[p, e, sl] = rows[p, e, sl] * w
                return c2

            lax.fori_loop(0, _GE // 16, sbody, 0)

        for idxR, wR, out, gt in ((idxA, wA, outA, 2 * ga),
                                  (idxK, wK, outK, 2 * gk)):
            ng0 = max(2, (gt // 6) // 2 * 2)

            for i in range(rps // _SB):
                pltpu.sync_copy(sbuf,
                                acc.at[pl.ds(r0 + i * _SB, _SB)])
            if tail:
                @pl.when(s == _NS - 1)
                def _():
                    t0 = _NS * rps
                    for i in range(tail // _SB):
                        pltpu.sync_copy(sbuf,
                                        acc.at[pl.ds(t0 + i * _SB, _SB)])
                    rem = tail % _SB
                    if rem:
                        pltpu.sync_copy(
                            sbuf.at[pl.ds(0, rem)],
                            acc.at[pl.ds(t0 + (tail // _SB) * _SB, rem)])
            plsc.subcore_barrier()

            def pipe(ng, base_g, idxR=idxR, wR=wR):
                def idx_dma(g, p):
                    cb = (base_g + g) * _GRP * 2
                    wb = pl.multiple_of((base_g + g) * _GE, _GE)
                    return (
                        pltpu.make_async_copy(
                            idxR.at[pl.ds(cb, _GRP * 2)],
                            idxb.at[pl.ds(p * _GRP * 2, _GRP * 2)], semI),
                        pltpu.make_async_copy(wR.at[pl.ds(wb, _GE)],
                                              wbuf.at[p], semI),
                    )

                def gathers(p):
                    return [
                        pltpu.make_async_copy(
                            sup.at[idxb.at[p * _GRP * 2 + 2 * j]],
                            rows.at[p, pl.ds(j * _CHUNK, _CHUNK)], semG)
                        for j in range(_GRP)
                    ]

                def scatters(p):
                    return [
                        pltpu.make_async_copy(
                            rows.at[p, pl.ds(j * _CHUNK, _CHUNK)],
                            acc.at[idxb.at[p * _GRP * 2 + 2 * j + 1]], semS)
                        for j in range(_GRP)
                    ]

                def step(g, p, first=False, last=False):
                    if not first:
                        for dsc in scatters(p ^ 1):
                            dsc.wait()
                    if not last:
                        for dsc in idx_dma(g + 1, p ^ 1):
                            dsc.start()
                    for dsc in gathers(p):
                        dsc.wait()
                    scale(p)
                    for dsc in scatters(p):
                        dsc.start(add=True)
                    if not last:
                        for dsc in idx_dma(g + 1, p ^ 1):
                            dsc.wait()
                        for dsc in gathers(p ^ 1):
                            dsc.start()

                for dsc in idx_dma(0, 0):
                    dsc.start()
                for dsc in idx_dma(0, 0):
                    dsc.wait()
                for dsc in gathers(0):
                    dsc.start()
                step(0, 0, first=True)

                def pair(t, carry):
                    g = 1 + t * 2
                    step(g, 1)
                    step(g + 1, 0)
                    return carry

                lax.fori_loop(0, (ng - 2) // 2, pair, 0)

                step(ng - 1, 1, last=True)
                for dsc in scatters(1):
                    dsc.wait()

            @pl.when(c == 0)
            def _(gt=gt, ng0=ng0):
                pipe(ng0, s * gt)

            @pl.when(c == 1)
            def _(gt=gt, ng0=ng0):
                pipe(gt - ng0, s * gt + ng0)

            plsc.subcore_barrier()
            o0 = pl.multiple_of(c * n + r0, 8)
            for i in range(rps // _SB):
                pltpu.sync_copy(acc.at[pl.ds(r0 + i * _SB, _SB)], sbuf)
                pltpu.sync_copy(sbuf, out.at[pl.ds(o0 + i * _SB, _SB)])
            if tail:
                @pl.when(s == _NS - 1)
                def _(out=out):
                    t0 = _NS * rps
                    ot = pl.multiple_of(c * n + t0, 8)
                    for i in range(tail // _SB):
                        pltpu.sync_copy(acc.at[pl.ds(t0 + i * _SB, _SB)],
                                        sbuf)
                        pltpu.sync_copy(sbuf,
                                        out.at[pl.ds(ot + i * _SB, _SB)])
                    rem = tail % _SB
                    b0 = (tail // _SB) * _SB
                    if rem:
                        pltpu.sync_copy(acc.at[pl.ds(t0 + b0, rem)],
                                        sbuf.at[pl.ds(0, rem)])
                        pltpu.sync_copy(sbuf.at[pl.ds(0, rem)],
                                        out.at[pl.ds(ot + b0, rem)])
            zero_sbuf()

    return spmm


@functools.lru_cache(maxsize=None)
def _make_proj(n, f, blk):
    def body(x_ref, w_ref, brow_ref, o_ref):
        o_ref[...] = (jnp.dot(x_ref[...], w_ref[...],
                              preferred_element_type=jnp.float32)
                      + brow_ref[...])

    return pl.pallas_call(
        body,
        grid=(n // blk,),
        in_specs=[
            pl.BlockSpec((blk, f), lambda i: (i, 0)),
            pl.BlockSpec((f, 128), lambda i: (0, 0)),
            pl.BlockSpec((1, 128), lambda i: (0, 0)),
        ],
        out_specs=pl.BlockSpec((blk, 128), lambda i: (i, 0)),
        out_shape=jax.ShapeDtypeStruct((n, 128), jnp.float32),
    )


@functools.lru_cache(maxsize=None)
def _make_comb1(n, nhid, blk):
    def body(p1_ref, hA0, hA1, hK0, hK1, b1row, w2_ref, brow2, o_ref):
        p1 = p1_ref[...]
        sup1 = p1[:, :nhid]
        s = jax.nn.sigmoid(p1[:, nhid:nhid + 1])
        dk = p1[:, nhid + 1:nhid + 2]
        b1 = b1row[...]
        hA = hA0[...] + hA1[...] + b1
        hK = hK0[...] + hK1[...] + b1
        h = s * hA + (1.0 - s) * hK + _G * dk * (sup1 + b1)
        o_ref[...] = (jnp.dot(h, w2_ref[...],
                              preferred_element_type=jnp.float32)
                      + brow2[...])

    part = pl.BlockSpec((blk, nhid), lambda i: (i, 0))
    return pl.pallas_call(
        body,
        grid=(n // blk,),
        in_specs=[
            pl.BlockSpec((blk, 128), lambda i: (i, 0)),
            part, part, part, part,
            pl.BlockSpec((1, nhid), lambda i: (0, 0)),
            pl.BlockSpec((nhid, 128), lambda i: (0, 0)),
            pl.BlockSpec((1, 128), lambda i: (0, 0)),
        ],
        out_specs=pl.BlockSpec((blk, 128), lambda i: (i, 0)),
        out_shape=jax.ShapeDtypeStruct((n, 128), jnp.float32),
    )


@functools.lru_cache(maxsize=None)
def _make_comb2(n, ncls, blk):
    def body(p2_ref, oA0, oA1, oK0, oK1, b2row, o_ref):
        p2 = p2_ref[...]
        sup2 = p2[:, :ncls]
        s = jax.nn.sigmoid(p2[:, ncls:ncls + 1])
        dk = p2[:, ncls + 1:ncls + 2]
        b2 = b2row[...]
        oA = oA0[...] + oA1[...] + b2
        oK = oK0[...] + oK1[...] + b2
        o = s * oA + (1.0 - s) * oK + _G * dk * (sup2 + b2)
        m = jnp.max(o, axis=1, keepdims=True)
        lse = jnp.log(jnp.sum(jnp.exp(o - m), axis=1, keepdims=True)) + m
        o_ref[...] = o - lse

    part = pl.BlockSpec((blk, ncls), lambda i: (i, 0))
    return pl.pallas_call(
        body,
        grid=(n // blk,),
        in_specs=[
            pl.BlockSpec((blk, 128), lambda i: (i, 0)),
            part, part, part, part,
            pl.BlockSpec((1, ncls), lambda i: (0, 0)),
        ],
        out_specs=pl.BlockSpec((blk, ncls), lambda i: (i, 0)),
        out_shape=jax.ShapeDtypeStruct((n, ncls), jnp.float32),
    )


def kernel(x, edge_index, edge_weight, knn_edge_index, knn_edge_weight,
           W1, b1, W2, b2, scores0, scores1, bias0, bias1,
           D_k0, D_k1, D_bias0, D_bias1):
    n, nfeat = x.shape
    nhid = W1.shape[1]
    ncls = W2.shape[1]
    blk = 2000

    mult = _NW * _GE * 2
    iA, wA, epa = _pad_edges(edge_index[1], edge_index[0],
                             edge_weight, mult)
    iK, wK, epk = _pad_edges(knn_edge_index[1], knn_edge_index[0],
                             knn_edge_weight, mult)

    wcat1 = (jnp.zeros((nfeat, 128), jnp.float32)
             .at[:, :nhid].set(W1)
             .at[:, nhid].set(scores0[:, 0])
             .at[:, nhid + 1].set(D_k0[:, 0]))
    brow1 = (jnp.zeros((1, 128), jnp.float32)
             .at[0, nhid].set(bias0[0])
             .at[0, nhid + 1].set(D_bias0[0]))
    p1 = _make_proj(n, nfeat, blk)(x, wcat1, brow1)
    sup1 = p1[:, :nhid]

    nc = 2
    hA, hK = _make_spmm(n, nhid, epa, epk, nc)(sup1, iA, wA, iK, wK)

    def _parts(h):
        return (h[:n], h[n:]) if nc == 2 else (h, jnp.zeros_like(h))

    wcat2 = (jnp.zeros((nhid, 128), jnp.float32)
             .at[:, :ncls].set(W2)
             .at[:, ncls].set(scores1[:, 0])
             .at[:, ncls + 1].set(D_k1[:, 0]))
    brow2 = (jnp.zeros((1, 128), jnp.float32)
             .at[0, ncls].set(bias1[0])
             .at[0, ncls + 1].set(D_bias1[0]))
    p2 = _make_comb1(n, nhid, blk)(p1, *_parts(hA), *_parts(hK),
                                   b1[None, :], wcat2, brow2)
    sup2 = p2[:, :ncls]

    oA, oK = _make_spmm(n, ncls, epa, epk, nc)(sup2, iA, wA, iK, wK)

    return _make_comb2(n, ncls, blk)(p2, *_parts(oA), *_parts(oK),
                                     b2[None, :])


# speedup vs baseline: 1.0387x; 1.0387x over previous
import functools

import jax
import jax.numpy as jnp
from jax import lax
from jax.experimental import pallas as pl
from jax.experimental.pallas import tpu as pltpu
from jax.experimental.pallas import tpu_sc as plsc

_G = 0.1
_CHUNK = 128
_NC = 2
_NS = 16
_NW = _NC * _NS


_GRP = 4
_GE = _GRP * _CHUNK
_XTRA = 2 * _GRP


def _pad_edges(src, dst, w, mult):
    e = src.shape[0]
    ep = -(-e // mult) * mult
    src = jnp.concatenate([src, jnp.zeros((ep - e,), jnp.int32)])
    dst = jnp.concatenate([dst, jnp.zeros((ep - e,), jnp.int32)])
    w = jnp.concatenate([w, jnp.zeros((ep - e + _XTRA * _CHUNK,),
                                      jnp.float32)])
    idx = jnp.stack([src.reshape(-1, _CHUNK), dst.reshape(-1, _CHUNK)],
                    axis=1).reshape(-1, _CHUNK)
    idx = jnp.concatenate(
        [idx, jnp.zeros((2 * _XTRA, _CHUNK), jnp.int32)])
    return idx, w, ep


@functools.lru_cache(maxsize=None)
def _make_spmm(n, d, epa, epk, nc=_NC):
    nw = nc * _NS
    ga = epa // (nw * _GE)
    gk = epk // (nw * _GE)
    rps = (n // _NS) // 8 * 8
    tail = n - _NS * rps
    mesh = plsc.VectorSubcoreMesh(core_axis_name="c", subcore_axis_name="s",
                                  num_cores=nc)

    @functools.partial(
        pl.kernel,
        mesh=mesh,
        compiler_params=pltpu.CompilerParams(use_tc_tiling_on_sc=False),
        out_type=(jax.ShapeDtypeStruct((nc * n, d), jnp.float32),
                  jax.ShapeDtypeStruct((nc * n, d), jnp.float32)),
        scratch_types=[
            pltpu.VMEM((2 * _GRP * 2, _CHUNK), jnp.int32),
            pltpu.VMEM((2, _GE), jnp.float32),
            pltpu.VMEM((2, _GE, d), jnp.float32),
            pltpu.VMEM((208, d), jnp.float32),
            pltpu.VMEM_SHARED((n, d), jnp.float32),
            pltpu.SemaphoreType.DMA,
            pltpu.SemaphoreType.DMA,
            pltpu.SemaphoreType.DMA,
        ],
    )
    def spmm(sup, idxA, wA, idxK, wK,
             outA, outK, idxb, wbuf, rows, sbuf, acc, semI, semG, semS):
        c = lax.axis_index("c")
        s = lax.axis_index("s")
        wid = c * _NS + s

        r0 = pl.multiple_of(s * rps, 8)
        _SB = 208
        zv = jnp.zeros((16,), jnp.float32)

        def zero_sbuf():
            def zb(i, c2):
                for gcol in range(d // 16):
                    sbuf[i, pl.ds(gcol * 16, 16)] = zv
                return c2

            lax.fori_loop(0, _SB, zb, 0)

        zero_sbuf()

        def scale(p):
            def sbody(e16, c2):
                e0 = pl.multiple_of(e16 * 16, 16)
                wvec = wbuf[p, pl.ds(e0, 16)]
                for lane in range(16):
                    e = e0 + lane
                    w = wvec[lane]
                    for g in range(d // 16):
                        sl = pl.ds(g * 16, 16)
                        rows[p, e, sl] = rows[p, e, sl] * w
                return c2

            lax.fori_loop(0, _GE // 16, sbody, 0)

        for idxR, wR, out, gt in ((idxA, wA, outA, 2 * ga),
                                  (idxK, wK, outK, 2 * gk)):
            ng0 = max(2, (gt // 6) // 2 * 2)

            for i in range(rps // _SB):
                pltpu.sync_copy(sbuf,
                                acc.at[pl.ds(r0 + i * _SB, _SB)])
            if tail:
                @pl.when(s == _NS - 1)
                def _():
                    t0 = _NS * rps
                    for i in range(tail // _SB):
                        pltpu.sync_copy(sbuf,
                                        acc.at[pl.ds(t0 + i * _SB, _SB)])
                    rem = tail % _SB
                    if rem:
                        pltpu.sync_copy(
                            sbuf.at[pl.ds(0, rem)],
                            acc.at[pl.ds(t0 + (tail // _SB) * _SB, rem)])
            plsc.subcore_barrier()

            def pipe(ng, base_g, idxR=idxR, wR=wR):
                def idx_dma(g, p):
                    cb = (base_g + g) * _GRP * 2
                    wb = pl.multiple_of((base_g + g) * _GE, _GE)
                    return (
                        pltpu.make_async_copy(
                            idxR.at[pl.ds(cb, _GRP * 2)],
                            idxb.at[pl.ds(p * _GRP * 2, _GRP * 2)], semI),
                        pltpu.make_async_copy(wR.at[pl.ds(wb, _GE)],
                                              wbuf.at[p], semI),
                    )

                def gathers(p):
                    return [
                        pltpu.make_async_copy(
                            sup.at[idxb.at[p * _GRP * 2 + 2 * j]],
                            rows.at[p, pl.ds(j * _CHUNK, _CHUNK)], semG)
                        for j in range(_GRP)
                    ]

                def scatters(p):
                    return [
                        pltpu.make_async_copy(
                            rows.at[p, pl.ds(j * _CHUNK, _CHUNK)],
                            acc.at[idxb.at[p * _GRP * 2 + 2 * j + 1]], semS)
                        for j in range(_GRP)
                    ]

                def step(g, p, first=False, last=False):
                    if not first:
                        for dsc in scatters(p ^ 1):
                            dsc.wait()
                    if not last:
                        for dsc in idx_dma(g + 1, p ^ 1):
                            dsc.start()
                    for dsc in gathers(p):
                        dsc.wait()
                    scale(p)
                    for dsc in scatters(p):
                        dsc.start(add=True)
                    if not last:
                        for dsc in idx_dma(g + 1, p ^ 1):
                            dsc.wait()
                        for dsc in gathers(p ^ 1):
                            dsc.start()

                for dsc in idx_dma(0, 0):
                    dsc.start()
                for dsc in idx_dma(0, 0):
                    dsc.wait()
                for dsc in gathers(0):
                    dsc.start()
                step(0, 0, first=True)

                def pair(t, carry):
                    g = 1 + t * 2
                    step(g, 1)
                    step(g + 1, 0)
                    return carry

                lax.fori_loop(0, (ng - 2) // 2, pair, 0)

                step(ng - 1, 1, last=True)
                for dsc in scatters(1):
                    dsc.wait()

            @pl.when(c == 0)
            def _(gt=gt, ng0=ng0):
                pipe(gt - ng0, s * gt + ng0)

            @pl.when(c == 1)
            def _(gt=gt, ng0=ng0):
                pipe(ng0, s * gt)

            plsc.subcore_barrier()
            o0 = pl.multiple_of(c * n + r0, 8)
            for i in range(rps // _SB):
                pltpu.sync_copy(acc.at[pl.ds(r0 + i * _SB, _SB)], sbuf)
                pltpu.sync_copy(sbuf, out.at[pl.ds(o0 + i * _SB, _SB)])
            if tail:
                @pl.when(s == _NS - 1)
                def _(out=out):
                    t0 = _NS * rps
                    ot = pl.multiple_of(c * n + t0, 8)
                    for i in range(tail // _SB):
                        pltpu.sync_copy(acc.at[pl.ds(t0 + i * _SB, _SB)],
                                        sbuf)
                        pltpu.sync_copy(sbuf,
                                        out.at[pl.ds(ot + i * _SB, _SB)])
                    rem = tail % _SB
                    b0 = (tail // _SB) * _SB
                    if rem:
                        pltpu.sync_copy(acc.at[pl.ds(t0 + b0, rem)],
                                        sbuf.at[pl.ds(0, rem)])
                        pltpu.sync_copy(sbuf.at[pl.ds(0, rem)],
                                        out.at[pl.ds(ot + b0, rem)])
            zero_sbuf()

    return spmm


@functools.lru_cache(maxsize=None)
def _make_proj(n, f, blk):
    def body(x_ref, w_ref, brow_ref, o_ref):
        o_ref[...] = (jnp.dot(x_ref[...], w_ref[...],
                              preferred_element_type=jnp.float32)
                      + brow_ref[...])

    return pl.pallas_call(
        body,
        grid=(n // blk,),
        in_specs=[
            pl.BlockSpec((blk, f), lambda i: (i, 0)),
            pl.BlockSpec((f, 128), lambda i: (0, 0)),
            pl.BlockSpec((1, 128), lambda i: (0, 0)),
        ],
        out_specs=pl.BlockSpec((blk, 128), lambda i: (i, 0)),
        out_shape=jax.ShapeDtypeStruct((n, 128), jnp.float32),
    )


@functools.lru_cache(maxsize=None)
def _make_comb1(n, nhid, blk):
    def body(p1_ref, hA0, hA1, hK0, hK1, b1row, w2_ref, brow2, o_ref):
        p1 = p1_ref[...]
        sup1 = p1[:, :nhid]
        s = jax.nn.sigmoid(p1[:, nhid:nhid + 1])
        dk = p1[:, nhid + 1:nhid + 2]
        b1 = b1row[...]
        hA = hA0[...] + hA1[...] + b1
        hK = hK0[...] + hK1[...] + b1
        h = s * hA + (1.0 - s) * hK + _G * dk * (sup1 + b1)
        o_ref[...] = (jnp.dot(h, w2_ref[...],
                              preferred_element_type=jnp.float32)
                      + brow2[...])

    part = pl.BlockSpec((blk, nhid), lambda i: (i, 0))
    return pl.pallas_call(
        body,
        grid=(n // blk,),
        in_specs=[
            pl.BlockSpec((blk, 128), lambda i: (i, 0)),
            part, part, part, part,
            pl.BlockSpec((1, nhid), lambda i: (0, 0)),
            pl.BlockSpec((nhid, 128), lambda i: (0, 0)),
            pl.BlockSpec((1, 128), lambda i: (0, 0)),
        ],
        out_specs=pl.BlockSpec((blk, 128), lambda i: (i, 0)),
        out_shape=jax.ShapeDtypeStruct((n, 128), jnp.float32),
    )


@functools.lru_cache(maxsize=None)
def _make_comb2(n, ncls, blk):
    def body(p2_ref, oA0, oA1, oK0, oK1, b2row, o_ref):
        p2 = p2_ref[...]
        sup2 = p2[:, :ncls]
        s = jax.nn.sigmoid(p2[:, ncls:ncls + 1])
        dk = p2[:, ncls + 1:ncls + 2]
        b2 = b2row[...]
        oA = oA0[...] + oA1[...] + b2
        oK = oK0[...] + oK1[...] + b2
        o = s * oA + (1.0 - s) * oK + _G * dk * (sup2 + b2)
        m = jnp.max(o, axis=1, keepdims=True)
        lse = jnp.log(jnp.sum(jnp.exp(o - m), axis=1, keepdims=True)) + m
        o_ref[...] = o - lse

    part = pl.BlockSpec((blk, ncls), lambda i: (i, 0))
    return pl.pallas_call(
        body,
        grid=(n // blk,),
        in_specs=[
            pl.BlockSpec((blk, 128), lambda i: (i, 0)),
            part, part, part, part,
            pl.BlockSpec((1, ncls), lambda i: (0, 0)),
        ],
        out_specs=pl.BlockSpec((blk, ncls), lambda i: (i, 0)),
        out_shape=jax.ShapeDtypeStruct((n, ncls), jnp.float32),
    )


def kernel(x, edge_index, edge_weight, knn_edge_index, knn_edge_weight,
           W1, b1, W2, b2, scores0, scores1, bias0, bias1,
           D_k0, D_k1, D_bias0, D_bias1):
    n, nfeat = x.shape
    nhid = W1.shape[1]
    ncls = W2.shape[1]
    blk = 2000

    mult = _NW * _GE * 2
    iA, wA, epa = _pad_edges(edge_index[1], edge_index[0],
                             edge_weight, mult)
    iK, wK, epk = _pad_edges(knn_edge_index[1], knn_edge_index[0],
                             knn_edge_weight, mult)

    wcat1 = (jnp.zeros((nfeat, 128), jnp.float32)
             .at[:, :nhid].set(W1)
             .at[:, nhid].set(scores0[:, 0])
             .at[:, nhid + 1].set(D_k0[:, 0]))
    brow1 = (jnp.zeros((1, 128), jnp.float32)
             .at[0, nhid].set(bias0[0])
             .at[0, nhid + 1].set(D_bias0[0]))
    p1 = _make_proj(n, nfeat, blk)(x, wcat1, brow1)
    sup1 = p1[:, :nhid]

    nc = 2
    hA, hK = _make_spmm(n, nhid, epa, epk, nc)(sup1, iA, wA, iK, wK)

    def _parts(h):
        return (h[:n], h[n:]) if nc == 2 else (h, jnp.zeros_like(h))

    wcat2 = (jnp.zeros((nhid, 128), jnp.float32)
             .at[:, :ncls].set(W2)
             .at[:, ncls].set(scores1[:, 0])
             .at[:, ncls + 1].set(D_k1[:, 0]))
    brow2 = (jnp.zeros((1, 128), jnp.float32)
             .at[0, ncls].set(bias1[0])
             .at[0, ncls + 1].set(D_bias1[0]))
    p2 = _make_comb1(n, nhid, blk)(p1, *_parts(hA), *_parts(hK),
                                   b1[None, :], wcat2, brow2)
    sup2 = p2[:, :ncls]

    oA, oK = _make_spmm(n, ncls, epa, epk, nc)(sup2, iA, wA, iK, wK)

    return _make_comb2(n, ncls, blk)(p2, *_parts(oA), *_parts(oK),
                                     b2[None, :])


# speedup vs baseline: 1.3776x; 1.3263x over previous
import functools

import jax
import jax.numpy as jnp
from jax import lax
from jax.experimental import pallas as pl
from jax.experimental.pallas import tpu as pltpu
from jax.experimental.pallas import tpu_sc as plsc

_G = 0.1
_CHUNK = 128
_NC = 2
_NS = 16
_NW = _NC * _NS


_GRP = 4
_GE = _GRP * _CHUNK
_XTRA = 2 * _GRP


def _pad_edges(src, dst, w, mult):
    e = src.shape[0]
    ep = -(-e // mult) * mult
    src = jnp.concatenate([src, jnp.zeros((ep - e,), jnp.int32)])
    dst = jnp.concatenate([dst, jnp.zeros((ep - e,), jnp.int32)])
    w = jnp.concatenate([w, jnp.zeros((ep - e + _XTRA * _CHUNK,),
                                      jnp.float32)])
    idx = jnp.stack([src.reshape(-1, _CHUNK), dst.reshape(-1, _CHUNK)],
                    axis=1).reshape(-1, _CHUNK)
    idx = jnp.concatenate(
        [idx, jnp.zeros((2 * _XTRA, _CHUNK), jnp.int32)])
    return idx, w, ep


@functools.lru_cache(maxsize=None)
def _make_spmm(n, d, epa, epk, nc=_NC):
    nw = nc * _NS
    ga = epa // (nw * _GE)
    gk = epk // (nw * _GE)
    rps = (n // _NS) // 8 * 8
    tail = n - _NS * rps
    mesh = plsc.VectorSubcoreMesh(core_axis_name="c", subcore_axis_name="s",
                                  num_cores=nc)

    @functools.partial(
        pl.kernel,
        mesh=mesh,
        compiler_params=pltpu.CompilerParams(use_tc_tiling_on_sc=False),
        out_type=(jax.ShapeDtypeStruct((nc * n, d), jnp.float32),
                  jax.ShapeDtypeStruct((nc * n, d), jnp.float32)),
        scratch_types=[
            pltpu.VMEM((2 * _GRP * 2, _CHUNK), jnp.int32),
            pltpu.VMEM((2, _GE), jnp.float32),
            pltpu.VMEM((2, _GE, d), jnp.float32),
            pltpu.VMEM((208, d), jnp.float32),
            pltpu.VMEM_SHARED((n, d), jnp.float32),
            pltpu.SemaphoreType.DMA,
            pltpu.SemaphoreType.DMA,
            pltpu.SemaphoreType.DMA,
        ],
    )
    def spmm(sup, idxA, wA, idxK, wK,
             outA, outK, idxb, wbuf, rows, sbuf, acc, semI, semG, semS):
        c = lax.axis_index("c")
        s = lax.axis_index("s")
        wid = c * _NS + s

        r0 = pl.multiple_of(s * rps, 8)
        _SB = 208
        zv = jnp.zeros((16,), jnp.float32)

        def zero_sbuf():
            def zb(i, c2):
                for gcol in range(d // 16):
                    sbuf[i, pl.ds(gcol * 16, 16)] = zv
                return c2

            lax.fori_loop(0, _SB, zb, 0)

        zero_sbuf()

        def scale(p):
            @plsc.parallel_loop(0, _GE // 16, unroll=2)
            def _(e16):
                e0 = pl.multiple_of(e16 * 16, 16)
                wvec = wbuf[p, pl.ds(e0, 16)]
                blk = rows.at[p, pl.ds(e0, 16)]
                for lane in range(16):
                    w = wvec[lane]
                    for g in range(d // 16):
                        sl = pl.ds(g * 16, 16)
                        blk[lane, sl] = blk[lane, sl] * w

        for idxR, wR, out, gt in ((idxA, wA, outA, 2 * ga),
                                  (idxK, wK, outK, 2 * gk)):
            ng0 = gt // 2

            for i in range(rps // _SB):
                pltpu.sync_copy(sbuf,
                                acc.at[pl.ds(r0 + i * _SB, _SB)])
            if tail:
                @pl.when(s == _NS - 1)
                def _():
                    t0 = _NS * rps
                    for i in range(tail // _SB):
                        pltpu.sync_copy(sbuf,
                                        acc.at[pl.ds(t0 + i * _SB, _SB)])
                    rem = tail % _SB
                    if rem:
                        pltpu.sync_copy(
                            sbuf.at[pl.ds(0, rem)],
                            acc.at[pl.ds(t0 + (tail // _SB) * _SB, rem)])
            plsc.subcore_barrier()

            def pipe(ng, base_g, idxR=idxR, wR=wR):
                def idx_dma(g, p):
                    cb = (base_g + g) * _GRP * 2
                    wb = pl.multiple_of((base_g + g) * _GE, _GE)
                    return (
                        pltpu.make_async_copy(
                            idxR.at[pl.ds(cb, _GRP * 2)],
                            idxb.at[pl.ds(p * _GRP * 2, _GRP * 2)], semI),
                        pltpu.make_async_copy(wR.at[pl.ds(wb, _GE)],
                                              wbuf.at[p], semI),
                    )

                def gathers(p):
                    return [
                        pltpu.make_async_copy(
                            sup.at[idxb.at[p * _GRP * 2 + 2 * j]],
                            rows.at[p, pl.ds(j * _CHUNK, _CHUNK)], semG)
                        for j in range(_GRP)
                    ]

                def scatters(p):
                    return [
                        pltpu.make_async_copy(
                            rows.at[p, pl.ds(j * _CHUNK, _CHUNK)],
                            acc.at[idxb.at[p * _GRP * 2 + 2 * j + 1]], semS)
                        for j in range(_GRP)
                    ]

                def step(g, p, first=False, last=False):
                    if not first:
                        for dsc in scatters(p ^ 1):
                            dsc.wait()
                    if not last:
                        for dsc in idx_dma(g + 1, p ^ 1):
                            dsc.start()
                    for dsc in gathers(p):
                        dsc.wait()
                    scale(p)
                    for dsc in scatters(p):
                        dsc.start(add=True)
                    if not last:
                        for dsc in idx_dma(g + 1, p ^ 1):
                            dsc.wait()
                        for dsc in gathers(p ^ 1):
                            dsc.start()

                for dsc in idx_dma(0, 0):
                    dsc.start()
                for dsc in idx_dma(0, 0):
                    dsc.wait()
                for dsc in gathers(0):
                    dsc.start()
                step(0, 0, first=True)

                def pair(t, carry):
                    g = 1 + t * 2
                    step(g, 1)
                    step(g + 1, 0)
                    return carry

                lax.fori_loop(0, (ng - 2) // 2, pair, 0)

                step(ng - 1, 1, last=True)
                for dsc in scatters(1):
                    dsc.wait()

            @pl.when(c == 0)
            def _(gt=gt, ng0=ng0):
                pipe(gt - ng0, s * gt + ng0)

            @pl.when(c == 1)
            def _(gt=gt, ng0=ng0):
                pipe(ng0, s * gt)

            plsc.subcore_barrier()
            o0 = pl.multiple_of(c * n + r0, 8)
            for i in range(rps // _SB):
                pltpu.sync_copy(acc.at[pl.ds(r0 + i * _SB, _SB)], sbuf)
                pltpu.sync_copy(sbuf, out.at[pl.ds(o0 + i * _SB, _SB)])
            if tail:
                @pl.when(s == _NS - 1)
                def _(out=out):
                    t0 = _NS * rps
                    ot = pl.multiple_of(c * n + t0, 8)
                    for i in range(tail // _SB):
                        pltpu.sync_copy(acc.at[pl.ds(t0 + i * _SB, _SB)],
                                        sbuf)
                        pltpu.sync_copy(sbuf,
                                        out.at[pl.ds(ot + i * _SB, _SB)])
                    rem = tail % _SB
                    b0 = (tail // _SB) * _SB
                    if rem:
                        pltpu.sync_copy(acc.at[pl.ds(t0 + b0, rem)],
                                        sbuf.at[pl.ds(0, rem)])
                        pltpu.sync_copy(sbuf.at[pl.ds(0, rem)],
                                        out.at[pl.ds(ot + b0, rem)])
            zero_sbuf()

    return spmm


@functools.lru_cache(maxsize=None)
def _make_proj(n, f, blk):
    def body(x_ref, w_ref, brow_ref, o_ref):
        o_ref[...] = (jnp.dot(x_ref[...], w_ref[...],
                              preferred_element_type=jnp.float32)
                      + brow_ref[...])

    return pl.pallas_call(
        body,
        grid=(n // blk,),
        in_specs=[
            pl.BlockSpec((blk, f), lambda i: (i, 0)),
            pl.BlockSpec((f, 128), lambda i: (0, 0)),
            pl.BlockSpec((1, 128), lambda i: (0, 0)),
        ],
        out_specs=pl.BlockSpec((blk, 128), lambda i: (i, 0)),
        out_shape=jax.ShapeDtypeStruct((n, 128), jnp.float32),
    )


@functools.lru_cache(maxsize=None)
def _make_comb1(n, nhid, blk):
    def body(p1_ref, hA0, hA1, hK0, hK1, b1row, w2_ref, brow2, o_ref):
        p1 = p1_ref[...]
        sup1 = p1[:, :nhid]
        s = jax.nn.sigmoid(p1[:, nhid:nhid + 1])
        dk = p1[:, nhid + 1:nhid + 2]
        b1 = b1row[...]
        hA = hA0[...] + hA1[...] + b1
        hK = hK0[...] + hK1[...] + b1
        h = s * hA + (1.0 - s) * hK + _G * dk * (sup1 + b1)
        o_ref[...] = (jnp.dot(h, w2_ref[...],
                              preferred_element_type=jnp.float32)
                      + brow2[...])

    part = pl.BlockSpec((blk, nhid), lambda i: (i, 0))
    return pl.pallas_call(
        body,
        grid=(n // blk,),
        in_specs=[
            pl.BlockSpec((blk, 128), lambda i: (i, 0)),
            part, part, part, part,
            pl.BlockSpec((1, nhid), lambda i: (0, 0)),
            pl.BlockSpec((nhid, 128), lambda i: (0, 0)),
            pl.BlockSpec((1, 128), lambda i: (0, 0)),
        ],
        out_specs=pl.BlockSpec((blk, 128), lambda i: (i, 0)),
        out_shape=jax.ShapeDtypeStruct((n, 128), jnp.float32),
    )


@functools.lru_cache(maxsize=None)
def _make_comb2(n, ncls, blk):
    def body(p2_ref, oA0, oA1, oK0, oK1, b2row, o_ref):
        p2 = p2_ref[...]
        sup2 = p2[:, :ncls]
        s = jax.nn.sigmoid(p2[:, ncls:ncls + 1])
        dk = p2[:, ncls + 1:ncls + 2]
        b2 = b2row[...]
        oA = oA0[...] + oA1[...] + b2
        oK = oK0[...] + oK1[...] + b2
        o = s * oA + (1.0 - s) * oK + _G * dk * (sup2 + b2)
        m = jnp.max(o, axis=1, keepdims=True)
        lse = jnp.log(jnp.sum(jnp.exp(o - m), axis=1, keepdims=True)) + m
        o_ref[...] = o - lse

    part = pl.BlockSpec((blk, ncls), lambda i: (i, 0))
    return pl.pallas_call(
        body,
        grid=(n // blk,),
        in_specs=[
            pl.BlockSpec((blk, 128), lambda i: (i, 0)),
            part, part, part, part,
            pl.BlockSpec((1, ncls), lambda i: (0, 0)),
        ],
        out_specs=pl.BlockSpec((blk, ncls), lambda i: (i, 0)),
        out_shape=jax.ShapeDtypeStruct((n, ncls), jnp.float32),
    )


def kernel(x, edge_index, edge_weight, knn_edge_index, knn_edge_weight,
           W1, b1, W2, b2, scores0, scores1, bias0, bias1,
           D_k0, D_k1, D_bias0, D_bias1):
    n, nfeat = x.shape
    nhid = W1.shape[1]
    ncls = W2.shape[1]
    blk = 2000

    mult = _NW * _GE * 2
    iA, wA, epa = _pad_edges(edge_index[1], edge_index[0],
                             edge_weight, mult)
    iK, wK, epk = _pad_edges(knn_edge_index[1], knn_edge_index[0],
                             knn_edge_weight, mult)

    wcat1 = (jnp.zeros((nfeat, 128), jnp.float32)
             .at[:, :nhid].set(W1)
             .at[:, nhid].set(scores0[:, 0])
             .at[:, nhid + 1].set(D_k0[:, 0]))
    brow1 = (jnp.zeros((1, 128), jnp.float32)
             .at[0, nhid].set(bias0[0])
             .at[0, nhid + 1].set(D_bias0[0]))
    p1 = _make_proj(n, nfeat, blk)(x, wcat1, brow1)
    sup1 = p1[:, :nhid]

    nc = 2
    hA, hK = _make_spmm(n, nhid, epa, epk, nc)(sup1, iA, wA, iK, wK)

    def _parts(h):
        return (h[:n], h[n:]) if nc == 2 else (h, jnp.zeros_like(h))

    wcat2 = (jnp.zeros((nhid, 128), jnp.float32)
             .at[:, :ncls].set(W2)
             .at[:, ncls].set(scores1[:, 0])
             .at[:, ncls + 1].set(D_k1[:, 0]))
    brow2 = (jnp.zeros((1, 128), jnp.float32)
             .at[0, ncls].set(bias1[0])
             .at[0, ncls + 1].set(D_bias1[0]))
    p2 = _make_comb1(n, nhid, blk)(p1, *_parts(hA), *_parts(hK),
                                   b1[None, :], wcat2, brow2)
    sup2 = p2[:, :ncls]

    oA, oK = _make_spmm(n, ncls, epa, epk, nc)(sup2, iA, wA, iK, wK)

    return _make_comb2(n, ncls, blk)(p2, *_parts(oA), *_parts(oK),
                                     b2[None, :])
